# baseline (speedup 1.0000x reference)
import functools

import jax
import jax.numpy as jnp
from jax import lax
from jax.experimental import pallas as pl
from jax.experimental.pallas import tpu as pltpu
from jax.experimental.pallas import tpu_sc as plsc

_NUM_CORES = 2
_NUM_SUBCORES = 16
_NUM_WORKERS = _NUM_CORES * _NUM_SUBCORES
_LANES = 16
_CHUNK = 128
_ROWS_PER_SLAB = 4


def _gather_kernel(b_per_w, d):
    mesh = plsc.VectorSubcoreMesh(core_axis_name="c", subcore_axis_name="s")
    n_chunks = b_per_w // _CHUNK
    slab_w = _ROWS_PER_SLAB * d

    @functools.partial(
        pl.kernel,
        out_type=jax.ShapeDtypeStruct((_NUM_WORKERS * b_per_w, d), jnp.float32),
        mesh=mesh,
        scratch_types=[
            pltpu.VMEM((b_per_w,), jnp.int32),
            pltpu.VMEM((b_per_w,), jnp.int32),
            pltpu.VMEM((_CHUNK, slab_w), jnp.float32),
            pltpu.VMEM((_CHUNK, d), jnp.float32),
            pltpu.SemaphoreType.DMA,
        ],
    )
    def k(x_hbm, slab_hbm, out_hbm, idx_v, quo_v, slab_v, stage_v, sem):
        wid = lax.axis_index("s") * _NUM_CORES + lax.axis_index("c")
        base = wid * b_per_w
        pltpu.sync_copy(x_hbm.at[pl.ds(base, b_per_w)], idx_v)

        def quo(g, _):
            quo_v[pl.ds(g * _LANES, _LANES)] = (
                idx_v[pl.ds(g * _LANES, _LANES)] >> 2
            )
            return 0

        lax.fori_loop(0, b_per_w // _LANES, quo, 0)

        def chunk(c, _):
            c0 = c * _CHUNK
            pltpu.async_copy(
                slab_hbm.at[quo_v.at[pl.ds(c0, _CHUNK)]], slab_v, sem
            ).wait()

            def extract(g, _):
                v = idx_v[pl.ds(c0 + g * _LANES, _LANES)]
                for j in range(_LANES):
                    kl = g * _LANES + j
                    r = (v[j] & 3) * d
                    stage_v[kl, pl.ds(0, _LANES)] = slab_v[kl, pl.ds(r, _LANES)]
                    stage_v[kl, pl.ds(_LANES, _LANES)] = slab_v[
                        kl, pl.ds(r + _LANES, _LANES)
                    ]
                return 0

            lax.fori_loop(0, _CHUNK // _LANES, extract, 0)
            pltpu.sync_copy(stage_v, out_hbm.at[pl.ds(base + c0, _CHUNK)])
            return 0

        lax.fori_loop(0, n_chunks, chunk, 0)

    return k


@jax.jit
def kernel(x, table):
    (b,) = x.shape
    v, d = table.shape
    b_per_w = b // _NUM_WORKERS
    slabs = table.reshape(v // _ROWS_PER_SLAB, _ROWS_PER_SLAB * d)
    return _gather_kernel(b_per_w, d)(x, slabs)


# speedup vs baseline: 1.6812x; 1.6812x over previous
import functools

import jax
import jax.numpy as jnp
from jax import lax
from jax.experimental import pallas as pl
from jax.experimental.pallas import tpu as pltpu
from jax.experimental.pallas import tpu_sc as plsc

_NUM_CORES = 2
_NUM_SUBCORES = 16
_NUM_WORKERS = _NUM_CORES * _NUM_SUBCORES
_LANES = 16


def _gather_kernel(b_per_w, d):
    mesh = plsc.VectorSubcoreMesh(core_axis_name="c", subcore_axis_name="s")

    @functools.partial(
        pl.kernel,
        out_type=jax.ShapeDtypeStruct((_NUM_WORKERS * b_per_w, d), jnp.float32),
        mesh=mesh,
        scratch_types=[
            pltpu.VMEM((b_per_w,), jnp.int32),
            pltpu.VMEM((b_per_w, d), jnp.float32),
            pltpu.SemaphoreType.DMA,
        ],
    )
    def k(x_hbm, table_hbm, out_hbm, idx_v, rows_v, sem):
        wid = lax.axis_index("s") * _NUM_CORES + lax.axis_index("c")
        base = wid * b_per_w
        pltpu.sync_copy(x_hbm.at[pl.ds(base, b_per_w)], idx_v)

        def issue(g, _):
            v = idx_v[pl.ds(g * _LANES, _LANES)]
            kk = g * _LANES
            for j in range(_LANES):
                pltpu.async_copy(
                    table_hbm.at[pl.ds(v[j], 1), :],
                    rows_v.at[pl.ds(kk + j, 1), :],
                    sem,
                )
            return 0

        lax.fori_loop(0, b_per_w // _LANES, issue, 0)

        pltpu.make_async_copy(
            out_hbm.at[pl.ds(base, b_per_w)], rows_v, sem
        ).wait()
        pltpu.sync_copy(rows_v, out_hbm.at[pl.ds(base, b_per_w)])

    return k


@jax.jit
def kernel(x, table):
    (b,) = x.shape
    _, d = table.shape
    b_per_w = b // _NUM_WORKERS
    return _gather_kernel(b_per_w, d)(x, table)
